# Initial kernel scaffold; baseline (speedup 1.0000x reference)
import functools

import jax
import jax.numpy as jnp
from jax import lax
from jax.experimental import pallas as pl
from jax.experimental.pallas import tpu as pltpu
from jax.experimental.pallas import tpu_sc as plsc

_N = 10000
_D = 128
_R = 4
_EPR = 80000

_NC = 2
_NS = 16
_RPC = _R // _NC
_EBLK = 128
_EPS = 5120
_NCHUNK = _EPS // _EBLK
_EPAD = _NS * _EPS
_RPS = _N // _NS
_ZR = 125
_LANES = 16
_GRP = _D // _LANES


def _sc_aggregate(src_flat, dst_flat, w_flat, x):
    mesh = plsc.VectorSubcoreMesh(core_axis_name="c", subcore_axis_name="s")

    @functools.partial(
        pl.kernel,
        out_type=jax.ShapeDtypeStruct((_R, _N, _D), jnp.float32),
        mesh=mesh,
        scratch_types=[
            pltpu.VMEM((_EBLK,), jnp.int32),
            pltpu.VMEM((_EBLK,), jnp.int32),
            pltpu.VMEM((_EBLK,), jnp.float32),
            pltpu.VMEM((_EBLK, _D), jnp.float32),
            pltpu.VMEM((_ZR, _D), jnp.float32),
            pltpu.VMEM_SHARED((_N, _D), jnp.float32),
            pltpu.SemaphoreType.DMA,
        ],
    )
    def body(src_hbm, dst_hbm, w_hbm, x_hbm, a_hbm,
             srci, dsti, wv, rows, zbuf, acc, sem):
        c = lax.axis_index("c")
        s = lax.axis_index("s")
        row0 = s * _RPS

        zeros16 = jnp.zeros((_LANES,), jnp.float32)

        def zfill(i, carry):
            for g in range(_GRP):
                zbuf[i, pl.ds(g * _LANES, _LANES)] = zeros16
            return carry

        lax.fori_loop(0, _ZR, zfill, 0)

        for rl in range(_RPC):
            r = c * _RPC + rl
            for z in range(_RPS // _ZR):
                pltpu.sync_copy(zbuf, acc.at[pl.ds(row0 + z * _ZR, _ZR)])
            plsc.subcore_barrier()

            ebase = r * _EPAD + s * _EPS

            def chunk(k, carry):
                off = pl.multiple_of(ebase + k * _EBLK, 8)
                pltpu.sync_copy(src_hbm.at[pl.ds(off, _EBLK)], srci)
                pltpu.sync_copy(dst_hbm.at[pl.ds(off, _EBLK)], dsti)
                pltpu.sync_copy(w_hbm.at[pl.ds(off, _EBLK)], wv)
                pltpu.async_copy(x_hbm.at[srci], rows, sem).wait()

                def scale(e, c2):
                    wvec = jnp.full((_LANES,), wv[e], jnp.float32)
                    for g in range(_GRP):
                        sl = pl.ds(g * _LANES, _LANES)
                        rows[e, sl] = rows[e, sl] * wvec
                    return c2

                lax.fori_loop(0, _EBLK, scale, 0)
                pltpu.sync_copy(rows, acc.at[dsti], add=True)
                return carry

            lax.fori_loop(0, _NCHUNK, chunk, 0)
            plsc.subcore_barrier()

            for z in range(_RPS // _ZR):
                rsl = pl.ds(row0 + z * _ZR, _ZR)
                pltpu.sync_copy(acc.at[rsl], rows.at[pl.ds(0, _ZR)])
                pltpu.sync_copy(rows.at[pl.ds(0, _ZR)], a_hbm.at[r, rsl])

    return body(src_flat, dst_flat, w_flat, x)


_BN = 1000


def _tc_combine_body(x_ref, a_ref, wrel_ref, wroot_ref, b_ref, o_ref):
    acc = jnp.dot(x_ref[...], wroot_ref[...], preferred_element_type=jnp.float32)
    for r in range(_R):
        acc = acc + jnp.dot(a_ref[r], wrel_ref[r],
                            preferred_element_type=jnp.float32)
    o_ref[...] = jnp.maximum(acc + b_ref[...], 0.0)


def _tc_combine(x, a, w_rel, w_root, brow):
    return pl.pallas_call(
        _tc_combine_body,
        grid=(_N // _BN,),
        in_specs=[
            pl.BlockSpec((_BN, _D), lambda i: (i, 0)),
            pl.BlockSpec((_R, _BN, _D), lambda i: (0, i, 0)),
            pl.BlockSpec((_R, _D, _D), lambda i: (0, 0, 0)),
            pl.BlockSpec((_D, _D), lambda i: (0, 0)),
            pl.BlockSpec((1, _D), lambda i: (0, 0)),
        ],
        out_specs=pl.BlockSpec((_BN, _D), lambda i: (i, 0)),
        out_shape=jax.ShapeDtypeStruct((_N, _D), jnp.float32),
    )(x, a, w_rel, w_root, brow)


def _pad_stack(arrs, pad_value, dtype):
    parts = []
    for a in arrs:
        a2 = a.astype(dtype).reshape(_NS, _EPR // _NS)
        a2 = jnp.pad(a2, ((0, 0), (0, _EPS - _EPR // _NS)),
                     constant_values=pad_value)
        parts.append(a2.reshape(-1))
    return jnp.concatenate(parts)


def kernel(x, W_rel, W_root, b_root, bias,
           src_0, dst_0, w_0,
           src_1, dst_1, w_1,
           src_2, dst_2, w_2,
           src_3, dst_3, w_3):
    src_flat = _pad_stack([src_0, src_1, src_2, src_3], 0, jnp.int32)
    dst_flat = _pad_stack([dst_0, dst_1, dst_2, dst_3], 0, jnp.int32)
    w_flat = _pad_stack([w_0, w_1, w_2, w_3], 0.0, jnp.float32)

    a = _sc_aggregate(src_flat, dst_flat, w_flat, x)
    brow = (b_root + bias).reshape(1, _D)
    return _tc_combine(x, a, W_rel, W_root, brow)


# speedup vs baseline: 2.4020x; 2.4020x over previous
import functools

import jax
import jax.numpy as jnp
from jax import lax
from jax.experimental import pallas as pl
from jax.experimental.pallas import tpu as pltpu
from jax.experimental.pallas import tpu_sc as plsc

_N = 10000
_D = 128
_R = 4
_EPR = 80000

_NC = 2
_NS = 16
_RPC = _R // _NC
_EBLK = 128
_EPS = 5120
_NCHUNK = _EPS // _EBLK
_EPAD = _NS * _EPS
_NPAD = 10240
_RPS = _NPAD // _NS
_ZR = 128
_LANES = 16
_GRP = _D // _LANES


def _sc_aggregate(src_flat, dst_flat, w_flat, x):
    mesh = plsc.VectorSubcoreMesh(core_axis_name="c", subcore_axis_name="s")

    @functools.partial(
        pl.kernel,
        out_type=jax.ShapeDtypeStruct((_R, _NPAD, _D), jnp.float32),
        mesh=mesh,
        scratch_types=[
            pltpu.VMEM((_EBLK,), jnp.int32),
            pltpu.VMEM((_EBLK,), jnp.int32),
            pltpu.VMEM((_EBLK,), jnp.float32),
            pltpu.VMEM((_EBLK, _D), jnp.float32),
            pltpu.VMEM((_ZR, _D), jnp.float32),
            pltpu.VMEM_SHARED((_NPAD, _D), jnp.float32),
            pltpu.SemaphoreType.DMA,
        ],
    )
    def body(src_hbm, dst_hbm, w_hbm, x_hbm, a_hbm,
             srci, dsti, wv, rows, zbuf, acc, sem):
        c = lax.axis_index("c")
        s = lax.axis_index("s")
        row0 = s * _RPS

        zeros16 = jnp.zeros((_LANES,), jnp.float32)

        def zfill(i, carry):
            for g in range(_GRP):
                zbuf[i, pl.ds(g * _LANES, _LANES)] = zeros16
            return carry

        lax.fori_loop(0, _ZR, zfill, 0)

        for rl in range(_RPC):
            r = c * _RPC + rl
            for z in range(_RPS // _ZR):
                pltpu.sync_copy(zbuf, acc.at[pl.ds(row0 + z * _ZR, _ZR)])
            plsc.subcore_barrier()

            ebase = r * _EPAD + s * _EPS

            def chunk(k, carry):
                off = pl.multiple_of(ebase + k * _EBLK, 8)
                pltpu.sync_copy(src_hbm.at[pl.ds(off, _EBLK)], srci)
                pltpu.sync_copy(dst_hbm.at[pl.ds(off, _EBLK)], dsti)
                pltpu.sync_copy(w_hbm.at[pl.ds(off, _EBLK)], wv)
                pltpu.async_copy(x_hbm.at[srci], rows, sem).wait()

                def scale(e16, c2):
                    wvec = wv[pl.ds(e16 * _LANES, _LANES)]
                    for j in range(_LANES):
                        wb = jnp.full((_LANES,), wvec[j], jnp.float32)
                        e = e16 * _LANES + j
                        for g in range(_GRP):
                            sl = pl.ds(g * _LANES, _LANES)
                            rows[e, sl] = rows[e, sl] * wb
                    return c2

                lax.fori_loop(0, _EBLK // _LANES, scale, 0)
                pltpu.sync_copy(rows, acc.at[dsti], add=True)
                return carry

            lax.fori_loop(0, _NCHUNK, chunk, 0)
            plsc.subcore_barrier()

            for z in range(_RPS // _ZR):
                rsl = pl.ds(row0 + z * _ZR, _ZR)
                pltpu.sync_copy(acc.at[rsl], rows.at[pl.ds(0, _ZR)])
                pltpu.sync_copy(rows.at[pl.ds(0, _ZR)], a_hbm.at[r, rsl])

    return body(src_flat, dst_flat, w_flat, x)


_BN = 1000


def _tc_combine_body(x_ref, a_ref, wrel_ref, wroot_ref, b_ref, o_ref):
    acc = jnp.dot(x_ref[...], wroot_ref[...], preferred_element_type=jnp.float32)
    for r in range(_R):
        acc = acc + jnp.dot(a_ref[r], wrel_ref[r],
                            preferred_element_type=jnp.float32)
    o_ref[...] = jnp.maximum(acc + b_ref[...], 0.0)


def _tc_combine(x, a, w_rel, w_root, brow):
    return pl.pallas_call(
        _tc_combine_body,
        grid=(_N // _BN,),
        in_specs=[
            pl.BlockSpec((_BN, _D), lambda i: (i, 0)),
            pl.BlockSpec((_R, _BN, _D), lambda i: (0, i, 0)),
            pl.BlockSpec((_R, _D, _D), lambda i: (0, 0, 0)),
            pl.BlockSpec((_D, _D), lambda i: (0, 0)),
            pl.BlockSpec((1, _D), lambda i: (0, 0)),
        ],
        out_specs=pl.BlockSpec((_BN, _D), lambda i: (i, 0)),
        out_shape=jax.ShapeDtypeStruct((_N, _D), jnp.float32),
    )(x, a, w_rel, w_root, brow)


def _pad_stack(arrs, pad_value, dtype):
    parts = []
    for a in arrs:
        a2 = a.astype(dtype).reshape(_NS, _EPR // _NS)
        a2 = jnp.pad(a2, ((0, 0), (0, _EPS - _EPR // _NS)),
                     constant_values=pad_value)
        parts.append(a2.reshape(-1))
    return jnp.concatenate(parts)


def kernel(x, W_rel, W_root, b_root, bias,
           src_0, dst_0, w_0,
           src_1, dst_1, w_1,
           src_2, dst_2, w_2,
           src_3, dst_3, w_3):
    src_flat = _pad_stack([src_0, src_1, src_2, src_3], 0, jnp.int32)
    dst_flat = _pad_stack([dst_0, dst_1, dst_2, dst_3], 0, jnp.int32)
    w_flat = _pad_stack([w_0, w_1, w_2, w_3], 0.0, jnp.float32)

    a = _sc_aggregate(src_flat, dst_flat, w_flat, x)
    brow = (b_root + bias).reshape(1, _D)
    return _tc_combine(x, a, W_rel, W_root, brow)


# speedup vs baseline: 3.2112x; 1.3369x over previous
import functools

import jax
import jax.numpy as jnp
from jax import lax
from jax.experimental import pallas as pl
from jax.experimental.pallas import tpu as pltpu
from jax.experimental.pallas import tpu_sc as plsc

_N = 10000
_D = 128
_R = 4
_EPR = 80000

_NC = 2
_NS = 16
_RPC = _R // _NC
_EBLK = 128
_EPS = 5120
_NCHUNK = _EPS // _EBLK
_EPAD = _NS * _EPS
_NPAD = 10240
_RPS = _NPAD // _NS
_ZR = 128
_LANES = 16
_GRP = _D // _LANES


def _sc_aggregate(src_flat, dst_flat, w_flat, x):
    mesh = plsc.VectorSubcoreMesh(core_axis_name="c", subcore_axis_name="s")

    @functools.partial(
        pl.kernel,
        out_type=jax.ShapeDtypeStruct((_R, _NPAD, _D), jnp.float32),
        mesh=mesh,
        scratch_types=[
            pltpu.VMEM((_EPS,), jnp.int32),
            pltpu.VMEM((_EBLK,), jnp.int32),
            pltpu.VMEM((_EBLK,), jnp.int32),
            pltpu.VMEM((_EPS,), jnp.float32),
            pltpu.VMEM((_EBLK, _D), jnp.float32),
            pltpu.VMEM((_EBLK, _D), jnp.float32),
            pltpu.VMEM_SHARED((_NPAD, _D), jnp.float32),
            pltpu.SemaphoreType.DMA,
            pltpu.SemaphoreType.DMA,
            pltpu.SemaphoreType.DMA,
            pltpu.SemaphoreType.DMA,
            pltpu.SemaphoreType.DMA,
            pltpu.SemaphoreType.DMA,
        ],
    )
    def body(src_hbm, dst_hbm, w_hbm, x_hbm, a_hbm,
             srci, dstb0, dstb1, wv, rows0, rows1, acc,
             gsem0, gsem1, ssem0, ssem1, dsem0, dsem1):
        c = lax.axis_index("c")
        s = lax.axis_index("s")
        row0 = s * _RPS
        zeros16 = jnp.zeros((_LANES,), jnp.float32)

        def zfill(i, carry):
            for g in range(_GRP):
                rows1[i, pl.ds(g * _LANES, _LANES)] = zeros16
            return carry

        rows = (rows0, rows1)
        dstb = (dstb0, dstb1)
        gsem = (gsem0, gsem1)
        ssem = (ssem0, ssem1)
        dsem = (dsem0, dsem1)

        for rl in range(_RPC):
            r = c * _RPC + rl
            lax.fori_loop(0, _ZR, zfill, 0)
            for z in range(_RPS // _ZR):
                pltpu.sync_copy(rows1, acc.at[pl.ds(row0 + z * _ZR, _ZR)])

            ebase = pl.multiple_of(r * _EPAD + s * _EPS, 8)
            pltpu.sync_copy(src_hbm.at[pl.ds(ebase, _EPS)], srci)
            pltpu.sync_copy(w_hbm.at[pl.ds(ebase, _EPS)], wv)
            plsc.subcore_barrier()

            def load_chunk(k, b):
                off = pl.multiple_of(ebase + k * _EBLK, 8)
                pltpu.async_copy(dst_hbm.at[pl.ds(off, _EBLK)], dstb[b], dsem[b])
                idx = srci.at[pl.ds(k * _EBLK, _EBLK)]
                pltpu.async_copy(x_hbm.at[idx], rows[b], gsem[b])

            def chunk_wait(b):
                pltpu.make_async_copy(dst_hbm.at[pl.ds(0, _EBLK)],
                                      dstb[b], dsem[b]).wait()
                pltpu.make_async_copy(x_hbm.at[srci.at[pl.ds(0, _EBLK)]],
                                      rows[b], gsem[b]).wait()

            load_chunk(0, 0)
            load_chunk(1, 1)

            def scale(b, k):
                def sc16(e16, c2):
                    wvec = wv[pl.ds(k * _EBLK + e16 * _LANES, _LANES)]
                    for j in range(_LANES):
                        wb = jnp.full((_LANES,), wvec[j], jnp.float32)
                        e = e16 * _LANES + j
                        for g in range(_GRP):
                            sl = pl.ds(g * _LANES, _LANES)
                            rows[b][e, sl] = rows[b][e, sl] * wb
                    return c2
                lax.fori_loop(0, _EBLK // _LANES, sc16, 0)

            def pair(p, carry):
                k0 = 2 * p
                k1 = 2 * p + 1
                for b, k in ((0, k0), (1, k1)):
                    chunk_wait(b)
                    scale(b, k)
                    pltpu.async_copy(rows[b], acc.at[dstb[b]], ssem[b],
                                     add=True)
                for b, k in ((0, k0), (1, k1)):
                    pltpu.make_async_copy(rows[b], acc.at[dstb[b]],
                                          ssem[b]).wait()
                    load_chunk(lax.rem(k + 2, _NCHUNK), b)
                return carry

            lax.fori_loop(0, _NCHUNK // 2, pair, 0)
            chunk_wait(0)
            chunk_wait(1)
            plsc.subcore_barrier()

            for z in range(_RPS // _ZR):
                rsl = pl.ds(row0 + z * _ZR, _ZR)
                pltpu.sync_copy(acc.at[rsl], rows0.at[pl.ds(0, _ZR)])
                pltpu.sync_copy(rows0.at[pl.ds(0, _ZR)], a_hbm.at[r, rsl])

    return body(src_flat, dst_flat, w_flat, x)


_BN = 1000


def _tc_combine_body(x_ref, a_ref, wrel_ref, wroot_ref, b_ref, o_ref):
    acc = jnp.dot(x_ref[...], wroot_ref[...], preferred_element_type=jnp.float32)
    for r in range(_R):
        acc = acc + jnp.dot(a_ref[r], wrel_ref[r],
                            preferred_element_type=jnp.float32)
    o_ref[...] = jnp.maximum(acc + b_ref[...], 0.0)


def _tc_combine(x, a, w_rel, w_root, brow):
    return pl.pallas_call(
        _tc_combine_body,
        grid=(_N // _BN,),
        in_specs=[
            pl.BlockSpec((_BN, _D), lambda i: (i, 0)),
            pl.BlockSpec((_R, _BN, _D), lambda i: (0, i, 0)),
            pl.BlockSpec((_R, _D, _D), lambda i: (0, 0, 0)),
            pl.BlockSpec((_D, _D), lambda i: (0, 0)),
            pl.BlockSpec((1, _D), lambda i: (0, 0)),
        ],
        out_specs=pl.BlockSpec((_BN, _D), lambda i: (i, 0)),
        out_shape=jax.ShapeDtypeStruct((_N, _D), jnp.float32),
    )(x, a, w_rel, w_root, brow)


def _pad_stack(arrs, pad_value, dtype):
    parts = []
    for a in arrs:
        a2 = a.astype(dtype).reshape(_NS, _EPR // _NS)
        a2 = jnp.pad(a2, ((0, 0), (0, _EPS - _EPR // _NS)),
                     constant_values=pad_value)
        parts.append(a2.reshape(-1))
    return jnp.concatenate(parts)


def kernel(x, W_rel, W_root, b_root, bias,
           src_0, dst_0, w_0,
           src_1, dst_1, w_1,
           src_2, dst_2, w_2,
           src_3, dst_3, w_3):
    src_flat = _pad_stack([src_0, src_1, src_2, src_3], 0, jnp.int32)
    dst_flat = _pad_stack([dst_0, dst_1, dst_2, dst_3], 0, jnp.int32)
    w_flat = _pad_stack([w_0, w_1, w_2, w_3], 0.0, jnp.float32)

    a = _sc_aggregate(src_flat, dst_flat, w_flat, x)
    brow = (b_root + bias).reshape(1, _D)
    return _tc_combine(x, a, W_rel, W_root, brow)
